# baseline (speedup 1.0000x reference)
import functools

import jax
import jax.numpy as jnp
from jax import lax
from jax.experimental import pallas as pl
from jax.experimental.pallas import tpu as pltpu
from jax.experimental.pallas import tpu_sc as plsc

_D = 64
_S = 200
_B = 4096
_EPS = 1e-6
_N = _B * _S
_NW = 32
_BW = _B // _NW
_PER_W = _N // _NW
_CB = 16
_CS = 8
_CHUNK = _CB * _CS
_NSB = _S // _CS
_NCH = (_BW // _CB) * _NSB
_NBUF = 2
_L = 16


def _rsqrt(a):
    i = lax.bitcast_convert_type(a, jnp.int32)
    i = jnp.int32(0x5F3759DF) - lax.shift_right_logical(i, 1)
    y = lax.bitcast_convert_type(i, jnp.float32)
    ah = a * jnp.float32(0.5)
    y = y * (jnp.float32(1.5) - ah * y * y)
    y = y * (jnp.float32(1.5) - ah * y * y)
    y = y * (jnp.float32(1.5) - ah * y * y)
    return y


_mesh = plsc.VectorSubcoreMesh(core_axis_name="c", subcore_axis_name="s")


@functools.partial(
    pl.kernel,
    out_type=jax.ShapeDtypeStruct((_B, _S, _D), jnp.float32),
    mesh=_mesh,
    scratch_types=[
        pltpu.VMEM((_PER_W,), jnp.int32),
        pltpu.VMEM((104, 2 * _D), jnp.float32),
        pltpu.VMEM((2, _D), jnp.float32),
        pltpu.VMEM((_CHUNK,), jnp.int32),
        pltpu.VMEM((_CHUNK,), jnp.int32),
        pltpu.VMEM((_CHUNK,), jnp.int32),
        pltpu.VMEM((_CHUNK,), jnp.int32),
        pltpu.SMEM((_CHUNK,), jnp.int32),
        pltpu.SMEM((_CHUNK,), jnp.int32),
        pltpu.VMEM((_CHUNK, 2 * _D), jnp.float32),
        pltpu.VMEM((_CHUNK, 2 * _D), jnp.float32),
        pltpu.VMEM((_CB, _CS, _D), jnp.float32),
        pltpu.VMEM((_CB, _CS, _D), jnp.float32),
        pltpu.SemaphoreType.DMA,
        pltpu.SemaphoreType.DMA,
        pltpu.SemaphoreType.DMA,
        pltpu.SemaphoreType.DMA,
    ],
    compiler_params=pltpu.CompilerParams(needs_layout_passes=False),
)
def _emb(ids_hbm, tok_hbm, pos_hbm, scale_hbm, bias_hbm, out_hbm,
         ids_v, pos_v, sb_v, ix0, ix1, pv0, pv1, ps0, ps1, in0, in1,
         ot0, ot1, sg0, sg1, so0, so1):
    wid = lax.axis_index("s") * 2 + lax.axis_index("c")
    ix = (ix0, ix1)
    pv = (pv0, pv1)
    ps = (ps0, ps1)
    in_bufs = (in0, in1)
    out_bufs = (ot0, ot1)
    sem_g = (sg0, sg1)
    sem_o = (so0, so1)

    pltpu.sync_copy(ids_hbm.at[pl.ds(wid * _PER_W, _PER_W)], ids_v)
    pltpu.sync_copy(pos_hbm.at[pl.ds(0, 104)], pos_v)
    pltpu.sync_copy(scale_hbm, sb_v.at[0])
    pltpu.sync_copy(bias_hbm, sb_v.at[1])

    scale_r = [sb_v[0, pl.ds(_L * j, _L)] for j in range(_D // _L)]
    bias_r = [sb_v[1, pl.ds(_L * j, _L)] for j in range(_D // _L)]
    iota = lax.iota(jnp.int32, _L)
    pat = (iota // 8) * jnp.int32(_S) + (iota % 8)

    def start_gather(g, b):
        bb = g // _NSB
        sc = g % _NSB
        c0 = bb * (_CB * _S) + sc * _CS
        for k in range(_CHUNK // _L):
            toks = plsc.load_gather(ids_v, [pat + (c0 + 400 * k)])
            ix[b][pl.ds(k * _L, _L)] = toks
        pltpu.async_copy(tok_hbm.at[ix[b]], in_bufs[b], sem_g[b])

    def compute(b, sc):
        buf_in = in_bufs[b]
        buf_out = out_bufs[b]
        par = ps[b]
        s0 = sc * _CS

        @plsc.parallel_loop(0, _CHUNK, unroll=4)
        def _row(r):
            b_ = r // _CS
            s_ = r % _CS
            s = s0 + s_
            prow = s // 2
            poff = (s % 2) * _D
            cvec = jnp.broadcast_to(r % _D, (_L,))
            x = [plsc.load_gather(buf_in, [iota + (_L * (j % 2)), cvec])
                 + pos_v[prow, pl.ds(poff + _L * j, _L)]
                 for j in range(_D // _L)]
            ssum = (x[0] + x[1]) + (x[2] + x[3])
            mean = jnp.broadcast_to(
                jnp.sum(ssum) * jnp.float32(1.0 / _D), (_L,))
            d = [xj - mean for xj in x]
            vsum = (d[0] * d[0] + d[1] * d[1]) + (d[2] * d[2] + d[3] * d[3])
            var = jnp.broadcast_to(
                jnp.sum(vsum) * jnp.float32(1.0 / _D), (_L,))
            rs = _rsqrt(var + jnp.float32(_EPS))
            for j in range(_D // _L):
                buf_out[b_, s_, pl.ds(_L * j, _L)] = (
                    d[j] * (rs * scale_r[j]) + bias_r[j])

    for b in range(_NBUF):
        start_gather(jnp.int32(b), b)

    def outer(m, carry):
        base = m * _NBUF
        for b in range(_NBUF):
            g = base + b
            pltpu.make_async_copy(tok_hbm.at[pl.ds(0, _CHUNK)], in_bufs[b],
                                  sem_g[b]).wait()

            @pl.when(g >= _NBUF)
            def _():
                pltpu.make_async_copy(
                    out_bufs[b],
                    out_hbm.at[pl.ds(0, _CB), pl.ds(0, _CS), :],
                    sem_o[b]).wait()

            bb = g // _NSB
            sc = g % _NSB
            compute(b, sc)
            b0 = wid * _BW + bb * _CB
            pltpu.async_copy(
                out_bufs[b],
                out_hbm.at[pl.ds(b0, _CB), pl.ds(sc * _CS, _CS), :],
                sem_o[b])

            @pl.when(g + _NBUF < _NCH)
            def _():
                start_gather(g + _NBUF, b)

        return carry

    lax.fori_loop(0, _NCH // _NBUF, outer, jnp.int32(0))

    for b in range(_NBUF):
        pltpu.make_async_copy(out_bufs[b],
                              out_hbm.at[pl.ds(0, _CB), pl.ds(0, _CS), :],
                              sem_o[b]).wait()


def kernel(input_ids, token_emb_w, pos_emb_w, ln_scale, ln_bias):
    ids = input_ids.reshape(_N)
    tok2 = jnp.pad(token_emb_w, ((0, 0), (0, _D)))
    pos2 = pos_emb_w.reshape(256, 2 * _D)
    return _emb(ids, tok2, pos2, ln_scale, ln_bias)


# speedup vs baseline: 1.4205x; 1.4205x over previous
import functools

import jax
import jax.numpy as jnp
from jax import lax
from jax.experimental import pallas as pl
from jax.experimental.pallas import tpu as pltpu
from jax.experimental.pallas import tpu_sc as plsc

_D = 64
_S = 200
_B = 4096
_EPS = 1e-6
_N = _B * _S
_NW = 32
_BW = _B // _NW
_PER_W = _N // _NW
_CB = 16
_CS = 8
_CHUNK = _CB * _CS
_NSB = _S // _CS
_NCH = (_BW // _CB) * _NSB
_NBUF = 2
_L = 16


def _rsqrt(a):
    i = lax.bitcast_convert_type(a, jnp.int32)
    i = jnp.int32(0x5F3759DF) - lax.shift_right_logical(i, 1)
    y = lax.bitcast_convert_type(i, jnp.float32)
    ah = a * jnp.float32(0.5)
    y = y * (jnp.float32(1.5) - ah * y * y)
    y = y * (jnp.float32(1.5) - ah * y * y)
    return y


_mesh = plsc.VectorSubcoreMesh(core_axis_name="c", subcore_axis_name="s")


@functools.partial(
    pl.kernel,
    out_type=jax.ShapeDtypeStruct((_B, _S, _D), jnp.float32),
    mesh=_mesh,
    scratch_types=[
        pltpu.VMEM((_PER_W,), jnp.int32),
        pltpu.VMEM((104, 2 * _D), jnp.float32),
        pltpu.VMEM((2, _D), jnp.float32),
        pltpu.VMEM((_CHUNK,), jnp.int32),
        pltpu.VMEM((_CHUNK,), jnp.int32),
        pltpu.VMEM((_CHUNK, 2 * _D), jnp.float32),
        pltpu.VMEM((_CHUNK, 2 * _D), jnp.float32),
        pltpu.VMEM((_CB, _CS, _D), jnp.float32),
        pltpu.VMEM((_CB, _CS, _D), jnp.float32),
        pltpu.SemaphoreType.DMA,
        pltpu.SemaphoreType.DMA,
        pltpu.SemaphoreType.DMA,
        pltpu.SemaphoreType.DMA,
    ],
    compiler_params=pltpu.CompilerParams(needs_layout_passes=False),
)
def _emb(ids_hbm, tok_hbm, pos_hbm, scale_hbm, bias_hbm, out_hbm,
         ids_v, pos_v, sb_v, ix0, ix1, in0, in1,
         ot0, ot1, sg0, sg1, so0, so1):
    wid = lax.axis_index("s") * 2 + lax.axis_index("c")
    ix = (ix0, ix1)
    in_bufs = (in0, in1)
    out_bufs = (ot0, ot1)
    sem_g = (sg0, sg1)
    sem_o = (so0, so1)

    pltpu.sync_copy(ids_hbm.at[pl.ds(wid * _PER_W, _PER_W)], ids_v)
    pltpu.sync_copy(pos_hbm.at[pl.ds(0, 104)], pos_v)
    pltpu.sync_copy(scale_hbm, sb_v.at[0])
    pltpu.sync_copy(bias_hbm, sb_v.at[1])

    scale_r = [sb_v[0, pl.ds(_L * j, _L)] for j in range(_D // _L)]
    bias_r = [sb_v[1, pl.ds(_L * j, _L)] for j in range(_D // _L)]
    iota = lax.iota(jnp.int32, _L)
    pat = (iota // 8) * jnp.int32(_S) + (iota % 8)

    def start_gather(g, b):
        bb = g // _NSB
        sc = g % _NSB
        c0 = bb * (_CB * _S) + sc * _CS
        for k in range(_CHUNK // _L):
            toks = plsc.load_gather(ids_v, [pat + (c0 + 400 * k)])
            ix[b][pl.ds(k * _L, _L)] = toks
        pltpu.async_copy(tok_hbm.at[ix[b]], in_bufs[b], sem_g[b])

    def compute(b, sc):
        buf_in = in_bufs[b]
        buf_out = out_bufs[b]
        s0 = sc * _CS

        @plsc.parallel_loop(0, _CHUNK, unroll=4)
        def _row(r):
            b_ = r // _CS
            s_ = r % _CS
            s = s0 + s_
            prow = s // 2
            poff = (s % 2) * _D
            x = [buf_in[r, pl.ds(_L * j, _L)]
                 + pos_v[prow, pl.ds(poff + _L * j, _L)]
                 for j in range(_D // _L)]
            ssum = (x[0] + x[1]) + (x[2] + x[3])
            mean = jnp.broadcast_to(
                jnp.sum(ssum) * jnp.float32(1.0 / _D), (_L,))
            d = [xj - mean for xj in x]
            vsum = (d[0] * d[0] + d[1] * d[1]) + (d[2] * d[2] + d[3] * d[3])
            var = jnp.broadcast_to(
                jnp.sum(vsum) * jnp.float32(1.0 / _D), (_L,))
            rs = _rsqrt(var + jnp.float32(_EPS))
            for j in range(_D // _L):
                buf_out[b_, s_, pl.ds(_L * j, _L)] = (
                    d[j] * (rs * scale_r[j]) + bias_r[j])

    for b in range(_NBUF):
        start_gather(jnp.int32(b), b)

    def outer(m, carry):
        base = m * _NBUF
        for b in range(_NBUF):
            g = base + b
            pltpu.make_async_copy(tok_hbm.at[pl.ds(0, _CHUNK)], in_bufs[b],
                                  sem_g[b]).wait()

            @pl.when(g >= _NBUF)
            def _():
                pltpu.make_async_copy(
                    out_bufs[b],
                    out_hbm.at[pl.ds(0, _CB), pl.ds(0, _CS), :],
                    sem_o[b]).wait()

            bb = g // _NSB
            sc = g % _NSB
            compute(b, sc)
            b0 = wid * _BW + bb * _CB
            pltpu.async_copy(
                out_bufs[b],
                out_hbm.at[pl.ds(b0, _CB), pl.ds(sc * _CS, _CS), :],
                sem_o[b])

            @pl.when(g + _NBUF < _NCH)
            def _():
                start_gather(g + _NBUF, b)

        return carry

    lax.fori_loop(0, _NCH // _NBUF, outer, jnp.int32(0))

    for b in range(_NBUF):
        pltpu.make_async_copy(out_bufs[b],
                              out_hbm.at[pl.ds(0, _CB), pl.ds(0, _CS), :],
                              sem_o[b]).wait()


def kernel(input_ids, token_emb_w, pos_emb_w, ln_scale, ln_bias):
    ids = input_ids.reshape(_N)
    tok2 = jnp.pad(token_emb_w, ((0, 0), (0, _D)))
    pos2 = pos_emb_w.reshape(256, 2 * _D)
    return _emb(ids, tok2, pos2, ln_scale, ln_bias)
